# Initial kernel scaffold; baseline (speedup 1.0000x reference)
import jax
import jax.numpy as jnp
from jax import lax
from jax.experimental import pallas as pl
from jax.experimental.pallas import tpu as pltpu
from jax.experimental.pallas import tpu_sc as plsc

_N = 10000
_E = 160000
_D = 256
_C = 32
_G = 64
_L = 4

_NC = 2
_NS = 16
_NW = _NC * _NS
_EPW = _E // _NW
_KCH = 40
_BCH = _EPW // _KCH
_NPS = _N // _NS
_VL = 16


def _sc_mesh():
    return plsc.VectorSubcoreMesh(
        core_axis_name="core", subcore_axis_name="subcore",
        num_cores=_NC, num_subcores=_NS)


def _hist_body(src_hbm, dst_hbm, out_hbm, sidx, didx, ho, hi):
    c = lax.axis_index("core")
    s = lax.axis_index("subcore")
    wid = c * _NS + s
    pltpu.sync_copy(src_hbm.at[pl.ds(wid * _EPW, _EPW)], sidx.at[pl.ds(0, _EPW)])
    pltpu.sync_copy(dst_hbm.at[pl.ds(wid * _EPW, _EPW)], didx.at[pl.ds(0, _EPW)])

    z16 = jnp.zeros((_VL,), jnp.float32)

    @pl.loop(0, _N // _VL)
    def _(j):
        ho[pl.ds(j * _VL, _VL)] = z16
        hi[pl.ds(j * _VL, _VL)] = z16

    def bump(idxv, hist, elig):
        cnt, last = plsc.scan_count(idxv, mask=elig)
        m = last if elig is None else (last & elig)
        plsc.addupdate_scatter(hist, [idxv], cnt.astype(jnp.float32), mask=m)

    nfull = _EPW // _VL

    @pl.loop(0, nfull)
    def _(j):
        bump(sidx[pl.ds(j * _VL, _VL)], ho, None)
        bump(didx[pl.ds(j * _VL, _VL)], hi, None)

    rem = _EPW - nfull * _VL
    elig = lax.iota(jnp.int32, _VL) < rem
    bump(sidx[pl.ds(nfull * _VL, _VL)], ho, elig)
    bump(didx[pl.ds(nfull * _VL, _VL)], hi, elig)

    pltpu.sync_copy(ho, out_hbm.at[wid])
    pltpu.sync_copy(hi, out_hbm.at[_NW + wid])


def _run_hist(src_flat, dst_flat):
    k = pl.kernel(
        _hist_body,
        out_type=jax.ShapeDtypeStruct((2 * _NW, _N), jnp.float32),
        mesh=_sc_mesh(),
        scratch_types=[
            pltpu.VMEM((_EPW + _VL,), jnp.int32),
            pltpu.VMEM((_EPW + _VL,), jnp.int32),
            pltpu.VMEM((_N,), jnp.float32),
            pltpu.VMEM((_N,), jnp.float32),
        ],
    )
    return k(src_flat, dst_flat)


def _msg_body(y_hbm, src_hbm, dst_hbm, zero_hbm, out_hbm,
              sidx, didx, rows, agg_sh, sem):
    c = lax.axis_index("core")
    s = lax.axis_index("subcore")
    wid = c * _NS + s
    pltpu.sync_copy(src_hbm.at[wid], sidx)
    pltpu.sync_copy(dst_hbm.at[wid], didx)
    pltpu.sync_copy(zero_hbm.at[pl.ds(s * _NPS, _NPS)],
                    agg_sh.at[pl.ds(s * _NPS, _NPS)])
    plsc.subcore_barrier()

    @pl.loop(0, _KCH)
    def _(j):
        pltpu.async_copy(y_hbm.at[sidx.at[j]], rows, sem).wait()
        pltpu.sync_copy(rows, agg_sh.at[didx.at[j]], add=True)

    plsc.subcore_barrier()
    pltpu.sync_copy(agg_sh.at[pl.ds(s * _NPS, _NPS)],
                    out_hbm.at[c].at[pl.ds(s * _NPS, _NPS)])


def _run_msg(y1p, src3, dst3, zeros):
    k = pl.kernel(
        _msg_body,
        out_type=jax.ShapeDtypeStruct((_NC, _N, _C), jnp.float32),
        mesh=_sc_mesh(),
        scratch_types=[
            pltpu.VMEM((_KCH, _BCH), jnp.int32),
            pltpu.VMEM((_KCH, _BCH), jnp.int32),
            pltpu.VMEM((_BCH, _C), jnp.float32),
            pltpu.VMEM_SHARED((_N, _C), jnp.float32),
            pltpu.SemaphoreType.DMA,
        ],
    )
    return k(y1p, src3, dst3, zeros)


def _tc1_body(x_ref, w1_ref, w2_ref, b_ref, hist_ref, yp_ref, z_ref, c_ref):
    xv = x_ref[...]
    hist = hist_ref[...]
    r = lax.broadcasted_iota(jnp.int32, (2 * _NW, 2), 0)
    col = lax.broadcasted_iota(jnp.int32, (2 * _NW, 2), 1)
    sel = jnp.where((r < _NW) == (col == 0), 1.0, 0.0).astype(jnp.float32)
    degs = lax.dot_general(hist, sel, (((0,), (0,)), ((), ())),
                           preferred_element_type=jnp.float32)
    a_col = lax.rsqrt(jnp.maximum(degs[:, 0:1], 1.0))
    c_ref[...] = lax.rsqrt(jnp.maximum(degs[:, 1:2], 1.0))
    yp_ref[...] = jnp.dot(xv, w1_ref[...],
                          preferred_element_type=jnp.float32) * a_col
    z_ref[...] = jnp.dot(xv, w2_ref[...],
                         preferred_element_type=jnp.float32) + b_ref[...]


def _run_tc1(x, w1, w2, b2, hist):
    return pl.pallas_call(
        _tc1_body,
        out_shape=[
            jax.ShapeDtypeStruct((_N, _C), jnp.float32),
            jax.ShapeDtypeStruct((_N, _C), jnp.float32),
            jax.ShapeDtypeStruct((_N, 1), jnp.float32),
        ],
    )(x, w1, w2, b2, hist)


def _tc2_body(aggp_ref, c_ref, z_ref, i_ref, wd_ref, bd_ref, o_ref):
    agg = aggp_ref[0] + aggp_ref[1]
    h = jax.nn.sigmoid(c_ref[...] * agg + z_ref[...])
    grow = lax.broadcasted_iota(jnp.int32, (_G, _N), 0)
    onehot_t = jnp.where(grow == i_ref[...], 1.0, 0.0).astype(jnp.float32)
    sums = jnp.dot(onehot_t, h, preferred_element_type=jnp.float32)
    cnt = jnp.dot(onehot_t, jnp.ones((_N, 1), jnp.float32),
                  preferred_element_type=jnp.float32)
    pooled = sums / jnp.maximum(cnt, 1.0)
    logits = jnp.dot(pooled, wd_ref[...],
                     preferred_element_type=jnp.float32) + bd_ref[...]
    m = jnp.max(logits, axis=1, keepdims=True)
    e = jnp.exp(logits - m)
    o_ref[...] = e / jnp.sum(e, axis=1, keepdims=True)


def _run_tc2(aggp, c_col, z, i_row, wd, bd2):
    return pl.pallas_call(
        _tc2_body,
        out_shape=jax.ShapeDtypeStruct((_G, _L), jnp.float32),
    )(aggp, c_col, z, i_row, wd, bd2)


def kernel(x, edge_index, i, W1, W2, b, Wd, bd):
    src_flat = edge_index[0]
    dst_flat = edge_index[1]
    src3 = src_flat.reshape(_NW, _KCH, _BCH)
    dst3 = dst_flat.reshape(_NW, _KCH, _BCH)
    zeros = jnp.zeros((_N, _C), jnp.float32)
    b2 = b.reshape(1, _C)
    bd2 = bd.reshape(1, _L)
    i_row = i.reshape(1, _N)

    hist = _run_hist(src_flat, dst_flat)
    y1p, z, c_col = _run_tc1(x, W1, W2, b2, hist)
    aggp = _run_msg(y1p, src3, dst3, zeros)
    return _run_tc2(aggp, c_col, z, i_row, Wd, bd2)


# speedup vs baseline: 26.0878x; 26.0878x over previous
import dataclasses

import jax
import jax.numpy as jnp
from jax import lax
from jax.experimental import pallas as pl
from jax.experimental.pallas import tpu as pltpu
from jax.experimental.pallas import tpu_sc as plsc

_N = 10000
_E = 160000
_D = 256
_C = 32
_G = 64
_L = 4

_NC = 2
_NS = 16
_NW = _NC * _NS
_EPW = _E // _NW
_KCH = 40
_BCH = _EPW // _KCH
_NPS = _N // _NS
_VL = 16


def _sc_params():
    cp = pltpu.CompilerParams(use_tc_tiling_on_sc=False)
    if "needs_layout_passes" in pltpu.CompilerParams.__dataclass_fields__:
        cp = dataclasses.replace(cp, needs_layout_passes=False)
    return cp


def _sc_mesh():
    return plsc.VectorSubcoreMesh(
        core_axis_name="core", subcore_axis_name="subcore",
        num_cores=_NC, num_subcores=_NS)


def _hist_body(src_hbm, dst_hbm, out_hbm, sidx, didx, ho, hi):
    c = lax.axis_index("core")
    s = lax.axis_index("subcore")
    wid = c * _NS + s
    pltpu.sync_copy(src_hbm.at[pl.ds(wid * _EPW, _EPW)], sidx.at[pl.ds(0, _EPW)])
    pltpu.sync_copy(dst_hbm.at[pl.ds(wid * _EPW, _EPW)], didx.at[pl.ds(0, _EPW)])

    z16 = jnp.zeros((_VL,), jnp.float32)

    @pl.loop(0, _N // _VL)
    def _(j):
        ho[pl.ds(j * _VL, _VL)] = z16
        hi[pl.ds(j * _VL, _VL)] = z16

    def bump(idxv, hist, elig):
        cnt, last = plsc.scan_count(idxv, mask=elig)
        m = last if elig is None else (last & elig)
        plsc.addupdate_scatter(hist, [idxv], cnt.astype(jnp.float32), mask=m)

    nfull = _EPW // _VL

    @pl.loop(0, nfull)
    def _(j):
        bump(sidx[pl.ds(j * _VL, _VL)], ho, None)
        bump(didx[pl.ds(j * _VL, _VL)], hi, None)

    rem = _EPW - nfull * _VL
    elig = lax.iota(jnp.int32, _VL) < rem
    bump(sidx[pl.ds(nfull * _VL, _VL)], ho, elig)
    bump(didx[pl.ds(nfull * _VL, _VL)], hi, elig)

    pltpu.sync_copy(ho, out_hbm.at[wid])
    pltpu.sync_copy(hi, out_hbm.at[_NW + wid])


def _run_hist(src_flat, dst_flat):
    k = pl.kernel(
        _hist_body,
        out_type=jax.ShapeDtypeStruct((2 * _NW, _N), jnp.float32),
        mesh=_sc_mesh(),
        compiler_params=_sc_params(),
        scratch_types=[
            pltpu.VMEM((_EPW + _VL,), jnp.int32),
            pltpu.VMEM((_EPW + _VL,), jnp.int32),
            pltpu.VMEM((_N,), jnp.float32),
            pltpu.VMEM((_N,), jnp.float32),
        ],
    )
    return k(src_flat, dst_flat)


def _msg_body(y_hbm, src_hbm, dst_hbm, zero_hbm, out_hbm,
              sidx, didx, rows, agg_sh, sem):
    c = lax.axis_index("core")
    s = lax.axis_index("subcore")
    wid = c * _NS + s
    pltpu.sync_copy(src_hbm.at[wid], sidx)
    pltpu.sync_copy(dst_hbm.at[wid], didx)
    slab = 640
    off = pl.multiple_of(jnp.minimum(s * slab, _N - slab), 8)
    pltpu.sync_copy(zero_hbm.at[pl.ds(off, slab)],
                    agg_sh.at[pl.ds(off, slab)])
    plsc.subcore_barrier()

    @pl.loop(0, _KCH)
    def _(j):
        pltpu.async_copy(y_hbm.at[sidx.at[j]], rows, sem).wait()
        pltpu.sync_copy(rows, agg_sh.at[didx.at[j]], add=True)

    plsc.subcore_barrier()
    pltpu.sync_copy(agg_sh.at[pl.ds(off, slab)],
                    out_hbm.at[c].at[pl.ds(off, slab)])


def _run_msg(y1p, src3, dst3, zeros):
    k = pl.kernel(
        _msg_body,
        out_type=jax.ShapeDtypeStruct((_NC, _N, _C), jnp.float32),
        mesh=_sc_mesh(),
        compiler_params=_sc_params(),
        scratch_types=[
            pltpu.VMEM((_KCH, _BCH), jnp.int32),
            pltpu.VMEM((_KCH, _BCH), jnp.int32),
            pltpu.VMEM((_BCH, _C), jnp.float32),
            pltpu.VMEM_SHARED((_N, _C), jnp.float32),
            pltpu.SemaphoreType.DMA,
        ],
    )
    return k(y1p, src3, dst3, zeros)


def _tc1_body(x_ref, w1_ref, w2_ref, b_ref, hist_ref, yp_ref, z_ref, c_ref):
    xv = x_ref[...]
    hist = hist_ref[...]
    r = lax.broadcasted_iota(jnp.int32, (2 * _NW, 2), 0)
    col = lax.broadcasted_iota(jnp.int32, (2 * _NW, 2), 1)
    sel = jnp.where((r < _NW) == (col == 0), 1.0, 0.0).astype(jnp.float32)
    degs = lax.dot_general(hist, sel, (((0,), (0,)), ((), ())),
                           preferred_element_type=jnp.float32)
    a_col = lax.rsqrt(jnp.maximum(degs[:, 0:1], 1.0))
    c_ref[...] = lax.rsqrt(jnp.maximum(degs[:, 1:2], 1.0))
    yp_ref[...] = jnp.dot(xv, w1_ref[...],
                          preferred_element_type=jnp.float32) * a_col
    z_ref[...] = jnp.dot(xv, w2_ref[...],
                         preferred_element_type=jnp.float32) + b_ref[...]


def _run_tc1(x, w1, w2, b2, hist):
    return pl.pallas_call(
        _tc1_body,
        out_shape=[
            jax.ShapeDtypeStruct((_N, _C), jnp.float32),
            jax.ShapeDtypeStruct((_N, _C), jnp.float32),
            jax.ShapeDtypeStruct((_N, 1), jnp.float32),
        ],
    )(x, w1, w2, b2, hist)


def _tc2_body(aggp_ref, c_ref, z_ref, i_ref, wd_ref, bd_ref, o_ref):
    agg = aggp_ref[0] + aggp_ref[1]
    h = jax.nn.sigmoid(c_ref[...] * agg + z_ref[...])
    grow = lax.broadcasted_iota(jnp.int32, (_G, _N), 0)
    onehot_t = jnp.where(grow == i_ref[...], 1.0, 0.0).astype(jnp.float32)
    sums = jnp.dot(onehot_t, h, preferred_element_type=jnp.float32)
    cnt = jnp.dot(onehot_t, jnp.ones((_N, 1), jnp.float32),
                  preferred_element_type=jnp.float32)
    pooled = sums / jnp.maximum(cnt, 1.0)
    logits = jnp.dot(pooled, wd_ref[...],
                     preferred_element_type=jnp.float32) + bd_ref[...]
    m = jnp.max(logits, axis=1, keepdims=True)
    e = jnp.exp(logits - m)
    o_ref[...] = e / jnp.sum(e, axis=1, keepdims=True)


def _run_tc2(aggp, c_col, z, i_row, wd, bd2):
    return pl.pallas_call(
        _tc2_body,
        out_shape=jax.ShapeDtypeStruct((_G, _L), jnp.float32),
    )(aggp, c_col, z, i_row, wd, bd2)


def kernel(x, edge_index, i, W1, W2, b, Wd, bd):
    src_flat = edge_index[0]
    dst_flat = edge_index[1]
    src3 = src_flat.reshape(_NW, _KCH, _BCH)
    dst3 = dst_flat.reshape(_NW, _KCH, _BCH)
    zeros = jnp.zeros((_N, _C), jnp.float32)
    b2 = b.reshape(1, _C)
    bd2 = bd.reshape(1, _L)
    i_row = i.reshape(1, _N)

    hist = _run_hist(src_flat, dst_flat)
    y1p, z, c_col = _run_tc1(x, W1, W2, b2, hist)
    aggp = _run_msg(y1p, src3, dst3, zeros)
    return _run_tc2(aggp, c_col, z, i_row, Wd, bd2)


# speedup vs baseline: 33.0260x; 1.2660x over previous
import dataclasses

import jax
import jax.numpy as jnp
from jax import lax
from jax.experimental import pallas as pl
from jax.experimental.pallas import tpu as pltpu
from jax.experimental.pallas import tpu_sc as plsc

_N = 10000
_E = 160000
_D = 256
_C = 32
_G = 64
_L = 4

_NC = 2
_NS = 16
_NW = _NC * _NS
_EPW = _E // _NW
_KCH = 40
_BCH = _EPW // _KCH
_VL = 16
_NBUF = 4
_LOOK = 2


def _sc_params():
    cp = pltpu.CompilerParams(use_tc_tiling_on_sc=False)
    if "needs_layout_passes" in pltpu.CompilerParams.__dataclass_fields__:
        cp = dataclasses.replace(cp, needs_layout_passes=False)
    return cp


def _sc_mesh():
    return plsc.VectorSubcoreMesh(
        core_axis_name="core", subcore_axis_name="subcore",
        num_cores=_NC, num_subcores=_NS)


def _hist_body(edge_hbm, out_hbm, sidx, didx, ho, hi):
    c = lax.axis_index("core")
    s = lax.axis_index("subcore")
    wid = c * _NS + s
    pltpu.sync_copy(edge_hbm.at[0].at[wid], sidx)
    pltpu.sync_copy(edge_hbm.at[1].at[wid], didx)

    z16 = jnp.zeros((_VL,), jnp.float32)

    @pl.loop(0, _N // _VL)
    def _(j):
        ho[pl.ds(j * _VL, _VL)] = z16
        hi[pl.ds(j * _VL, _VL)] = z16

    def bump(idxv, hist, elig):
        cnt, last = plsc.scan_count(idxv, mask=elig)
        m = last if elig is None else (last & elig)
        plsc.addupdate_scatter(hist, [idxv], cnt.astype(jnp.float32), mask=m)

    nfull = _BCH // _VL
    tail = nfull * _VL - (_BCH - _VL)
    elig = lax.iota(jnp.int32, _VL) >= tail

    @pl.loop(0, _KCH)
    def _(r):
        @pl.loop(0, nfull)
        def _(w):
            bump(sidx[r, pl.ds(w * _VL, _VL)], ho, None)
            bump(didx[r, pl.ds(w * _VL, _VL)], hi, None)
        bump(sidx[r, pl.ds(_BCH - _VL, _VL)], ho, elig)
        bump(didx[r, pl.ds(_BCH - _VL, _VL)], hi, elig)

    pltpu.sync_copy(ho, out_hbm.at[wid])
    pltpu.sync_copy(hi, out_hbm.at[_NW + wid])


def _run_hist(edge2):
    k = pl.kernel(
        _hist_body,
        out_type=jax.ShapeDtypeStruct((2 * _NW, _N), jnp.float32),
        mesh=_sc_mesh(),
        compiler_params=_sc_params(),
        scratch_types=[
            pltpu.VMEM((_KCH, _BCH), jnp.int32),
            pltpu.VMEM((_KCH, _BCH), jnp.int32),
            pltpu.VMEM((_N,), jnp.float32),
            pltpu.VMEM((_N,), jnp.float32),
        ],
    )
    return k(edge2)


def _msg_body(y_hbm, edge_hbm, zero_hbm, out_hbm,
              sidx, didx, rows, agg_sh, gs0, gs1, gs2, gs3, ss0, ss1, ss2, ss3):
    c = lax.axis_index("core")
    s = lax.axis_index("subcore")
    wid = c * _NS + s
    gsem = (gs0, gs1, gs2, gs3)
    ssem = (ss0, ss1, ss2, ss3)
    pltpu.sync_copy(edge_hbm.at[0].at[wid], sidx)
    pltpu.sync_copy(edge_hbm.at[1].at[wid], didx)
    slab = 640
    off = pl.multiple_of(jnp.minimum(s * slab, _N - slab), 8)
    pltpu.sync_copy(zero_hbm, agg_sh.at[pl.ds(off, slab)])
    plsc.subcore_barrier()

    def rslice(slot):
        return rows.at[pl.ds(slot * _BCH, _BCH)]

    def fire_gather(j, slot):
        pltpu.async_copy(y_hbm.at[sidx.at[j]], rslice(slot), gsem[slot])

    def wait_gather(j, slot):
        pltpu.make_async_copy(y_hbm.at[sidx.at[j]], rslice(slot),
                              gsem[slot]).wait()

    def fire_scatter(j, slot):
        pltpu.async_copy(rslice(slot), agg_sh.at[didx.at[j]], ssem[slot],
                         add=True)

    def wait_scatter(j, slot):
        pltpu.make_async_copy(rslice(slot), agg_sh.at[didx.at[j]],
                              ssem[slot]).wait()

    for p in range(_LOOK):
        fire_gather(p, p)

    @pl.loop(0, _KCH, step=_NBUF)
    def _(base):
        for b in range(_NBUF):
            j = base + b
            wait_gather(j, b)
            fire_scatter(j, b)
            nslot = (b + _LOOK) % _NBUF

            @pl.when(j + _LOOK < _KCH)
            def _():
                @pl.when(j - _LOOK >= 0)
                def _():
                    wait_scatter(j - _LOOK, nslot)
                fire_gather(j + _LOOK, nslot)

    for b in range(_NBUF):
        wait_scatter(_KCH - _NBUF + b, b)

    plsc.subcore_barrier()
    pltpu.sync_copy(agg_sh.at[pl.ds(off, slab)],
                    out_hbm.at[c].at[pl.ds(off, slab)])


def _run_msg(y1p, edge2, zeros):
    k = pl.kernel(
        _msg_body,
        out_type=jax.ShapeDtypeStruct((_NC, _N, _C), jnp.float32),
        mesh=_sc_mesh(),
        compiler_params=_sc_params(),
        scratch_types=[
            pltpu.VMEM((_KCH, _BCH), jnp.int32),
            pltpu.VMEM((_KCH, _BCH), jnp.int32),
            pltpu.VMEM((_NBUF * _BCH, _C), jnp.float32),
            pltpu.VMEM_SHARED((_N, _C), jnp.float32),
        ] + [pltpu.SemaphoreType.DMA] * (2 * _NBUF),
    )
    return k(y1p, edge2, zeros)


def _tc0_body(x_ref, w1_ref, w2_ref, b_ref, y1_ref, z_ref):
    xb = x_ref[...].astype(jnp.bfloat16)
    y1_ref[...] = jnp.dot(xb, w1_ref[...].astype(jnp.bfloat16),
                          preferred_element_type=jnp.float32)
    z_ref[...] = jnp.dot(xb, w2_ref[...].astype(jnp.bfloat16),
                         preferred_element_type=jnp.float32) + b_ref[...]


def _run_tc0(x, w1, w2, b2):
    return pl.pallas_call(
        _tc0_body,
        out_shape=[
            jax.ShapeDtypeStruct((_N, _C), jnp.float32),
            jax.ShapeDtypeStruct((_N, _C), jnp.float32),
        ],
    )(x, w1, w2, b2)


def _tc1_body(y1_ref, hist_ref, yp_ref, c_ref):
    hist = hist_ref[...]
    r = lax.broadcasted_iota(jnp.int32, (2 * _NW, 2), 0)
    col = lax.broadcasted_iota(jnp.int32, (2 * _NW, 2), 1)
    sel = jnp.where((r < _NW) == (col == 0), 1.0, 0.0).astype(jnp.float32)
    degs = lax.dot_general(hist, sel, (((0,), (0,)), ((), ())),
                           preferred_element_type=jnp.float32)
    a_col = lax.rsqrt(jnp.maximum(degs[:, 0:1], 1.0))
    c_ref[...] = lax.rsqrt(jnp.maximum(degs[:, 1:2], 1.0))
    yp_ref[...] = y1_ref[...] * a_col


def _run_tc1(y1, hist):
    return pl.pallas_call(
        _tc1_body,
        out_shape=[
            jax.ShapeDtypeStruct((_N, _C), jnp.float32),
            jax.ShapeDtypeStruct((_N, 1), jnp.float32),
        ],
    )(y1, hist)


def _tc2_body(aggp_ref, c_ref, z_ref, i_ref, wd_ref, bd_ref, o_ref):
    agg = aggp_ref[0] + aggp_ref[1]
    h = jax.nn.sigmoid(c_ref[...] * agg + z_ref[...])
    grow = lax.broadcasted_iota(jnp.int32, (_G, _N), 0)
    onehot_t = (grow == i_ref[...]).astype(jnp.bfloat16)
    sums = jnp.dot(onehot_t, h.astype(jnp.bfloat16),
                   preferred_element_type=jnp.float32)
    cnt = jnp.dot(onehot_t, jnp.ones((_N, 1), jnp.bfloat16),
                  preferred_element_type=jnp.float32)
    pooled = sums / jnp.maximum(cnt, 1.0)
    logits = jnp.dot(pooled, wd_ref[...],
                     preferred_element_type=jnp.float32) + bd_ref[...]
    m = jnp.max(logits, axis=1, keepdims=True)
    e = jnp.exp(logits - m)
    o_ref[...] = e / jnp.sum(e, axis=1, keepdims=True)


def _run_tc2(aggp, c_col, z, i_row, wd, bd2):
    return pl.pallas_call(
        _tc2_body,
        out_shape=jax.ShapeDtypeStruct((_G, _L), jnp.float32),
    )(aggp, c_col, z, i_row, wd, bd2)


def kernel(x, edge_index, i, W1, W2, b, Wd, bd):
    edge2 = edge_index.reshape(2, _NW, _KCH, _BCH)
    zeros = jnp.zeros((640, _C), jnp.float32)
    b2 = b.reshape(1, _C)
    bd2 = bd.reshape(1, _L)
    i_row = i.reshape(1, _N)

    hist = _run_hist(edge2)
    y1, z = _run_tc0(x, W1, W2, b2)
    y1p, c_col = _run_tc1(y1, hist)
    aggp = _run_msg(y1p, edge2, zeros)
    return _run_tc2(aggp, c_col, z, i_row, Wd, bd2)


# speedup vs baseline: 35.6004x; 1.0780x over previous
import dataclasses

import jax
import jax.numpy as jnp
from jax import lax
from jax.experimental import pallas as pl
from jax.experimental.pallas import tpu as pltpu
from jax.experimental.pallas import tpu_sc as plsc

_N = 10000
_E = 160000
_D = 256
_C = 32
_G = 64
_L = 4

_NC = 2
_NS = 16
_NW = _NC * _NS
_EPW = _E // _NW
_KCH = 40
_BCH = _EPW // _KCH
_VL = 16
_NBUF = 4
_LOOK = 2


def _sc_params():
    cp = pltpu.CompilerParams(use_tc_tiling_on_sc=False)
    if "needs_layout_passes" in pltpu.CompilerParams.__dataclass_fields__:
        cp = dataclasses.replace(cp, needs_layout_passes=False)
    return cp


def _sc_mesh():
    return plsc.VectorSubcoreMesh(
        core_axis_name="core", subcore_axis_name="subcore",
        num_cores=_NC, num_subcores=_NS)


def _hist_body(edge_hbm, out_hbm, sidx, didx, hp):
    c = lax.axis_index("core")
    s = lax.axis_index("subcore")
    wid = c * _NS + s
    pltpu.sync_copy(edge_hbm.at[0].at[wid], sidx)
    pltpu.sync_copy(edge_hbm.at[1].at[wid], didx)

    z16 = jnp.zeros((_VL,), jnp.int32)

    @pl.loop(0, _N // _VL)
    def _(j):
        hp[pl.ds(j * _VL, _VL)] = z16

    def bump(idxv, weight, elig):
        cnt, last = plsc.scan_count(idxv, mask=elig)
        m = last if elig is None else (last & elig)
        plsc.addupdate_scatter(hp, [idxv], cnt * weight, mask=m)

    nfull = _BCH // _VL
    tail = nfull * _VL - (_BCH - _VL)
    elig = lax.iota(jnp.int32, _VL) >= tail

    @pl.loop(0, _KCH)
    def _(r):
        for w in range(nfull):
            bump(sidx[r, pl.ds(w * _VL, _VL)], 1, None)
            bump(didx[r, pl.ds(w * _VL, _VL)], 65536, None)
        bump(sidx[r, pl.ds(_BCH - _VL, _VL)], 1, elig)
        bump(didx[r, pl.ds(_BCH - _VL, _VL)], 65536, elig)

    pltpu.sync_copy(hp, out_hbm.at[wid])


def _run_hist(edge2):
    k = pl.kernel(
        _hist_body,
        out_type=jax.ShapeDtypeStruct((_NW, _N), jnp.int32),
        mesh=_sc_mesh(),
        compiler_params=_sc_params(),
        scratch_types=[
            pltpu.VMEM((_KCH, _BCH), jnp.int32),
            pltpu.VMEM((_KCH, _BCH), jnp.int32),
            pltpu.VMEM((_N,), jnp.int32),
        ],
    )
    return k(edge2)


def _msg_body(y_hbm, edge_hbm, zero_hbm, out_hbm,
              sidx, didx, rows, agg_sh, gs0, gs1, gs2, gs3, ss0, ss1, ss2, ss3):
    c = lax.axis_index("core")
    s = lax.axis_index("subcore")
    wid = c * _NS + s
    gsem = (gs0, gs1, gs2, gs3)
    ssem = (ss0, ss1, ss2, ss3)
    pltpu.sync_copy(edge_hbm.at[0].at[wid], sidx)
    pltpu.sync_copy(edge_hbm.at[1].at[wid], didx)
    slab = 640
    off = pl.multiple_of(jnp.minimum(s * slab, _N - slab), 8)
    pltpu.sync_copy(zero_hbm, agg_sh.at[pl.ds(off, slab)])
    plsc.subcore_barrier()

    def rslice(slot):
        return rows.at[pl.ds(slot * _BCH, _BCH)]

    def fire_gather(j, slot):
        pltpu.async_copy(y_hbm.at[sidx.at[j]], rslice(slot), gsem[slot])

    def wait_gather(j, slot):
        pltpu.make_async_copy(y_hbm.at[sidx.at[j]], rslice(slot),
                              gsem[slot]).wait()

    def fire_scatter(j, slot):
        pltpu.async_copy(rslice(slot), agg_sh.at[didx.at[j]], ssem[slot],
                         add=True)

    def wait_scatter(j, slot):
        pltpu.make_async_copy(rslice(slot), agg_sh.at[didx.at[j]],
                              ssem[slot]).wait()

    for p in range(_LOOK):
        fire_gather(p, p)

    @pl.loop(0, _KCH, step=_NBUF)
    def _(base):
        for b in range(_NBUF):
            j = base + b
            wait_gather(j, b)
            fire_scatter(j, b)
            nslot = (b + _LOOK) % _NBUF

            @pl.when(j + _LOOK < _KCH)
            def _():
                @pl.when(j - _LOOK >= 0)
                def _():
                    wait_scatter(j - _LOOK, nslot)
                fire_gather(j + _LOOK, nslot)

    for b in range(_NBUF):
        wait_scatter(_KCH - _NBUF + b, b)

    plsc.subcore_barrier()
    pltpu.sync_copy(agg_sh.at[pl.ds(off, slab)],
                    out_hbm.at[c].at[pl.ds(off, slab)])


def _run_msg(y1p, edge2, zeros):
    k = pl.kernel(
        _msg_body,
        out_type=jax.ShapeDtypeStruct((_NC, _N, _C), jnp.bfloat16),
        mesh=_sc_mesh(),
        compiler_params=_sc_params(),
        scratch_types=[
            pltpu.VMEM((_KCH, _BCH), jnp.int32),
            pltpu.VMEM((_KCH, _BCH), jnp.int32),
            pltpu.VMEM((_NBUF * _BCH, _C), jnp.bfloat16),
            pltpu.VMEM_SHARED((_N, _C), jnp.bfloat16),
        ] + [pltpu.SemaphoreType.DMA] * (2 * _NBUF),
    )
    return k(y1p, edge2, zeros)


def _tc0_body(x_ref, w1_ref, w2_ref, b_ref, y1_ref, z_ref):
    xb = x_ref[...].astype(jnp.bfloat16)
    y1_ref[...] = jnp.dot(xb, w1_ref[...].astype(jnp.bfloat16),
                          preferred_element_type=jnp.float32
                          ).astype(jnp.bfloat16)
    z_ref[...] = (jnp.dot(xb, w2_ref[...].astype(jnp.bfloat16),
                          preferred_element_type=jnp.float32)
                  + b_ref[...]).astype(jnp.bfloat16)


def _run_tc0(x, w1, w2, b2):
    return pl.pallas_call(
        _tc0_body,
        out_shape=[
            jax.ShapeDtypeStruct((_N, _C), jnp.bfloat16),
            jax.ShapeDtypeStruct((_N, _C), jnp.bfloat16),
        ],
    )(x, w1, w2, b2)


def _tc1_body(y1_ref, hist_ref, yp_ref, c_ref):
    h = hist_ref[...]
    lo = (h & 0xFFFF).astype(jnp.float32)
    hi = lax.shift_right_logical(h, 16).astype(jnp.float32)
    ones = jnp.ones((_NW, 1), jnp.float32)
    tdims = (((0,), (0,)), ((), ()))
    dego = lax.dot_general(lo, ones, tdims,
                           preferred_element_type=jnp.float32)
    degi = lax.dot_general(hi, ones, tdims,
                           preferred_element_type=jnp.float32)
    a_col = lax.rsqrt(jnp.maximum(dego, 1.0))
    c_ref[...] = lax.rsqrt(jnp.maximum(degi, 1.0))
    yp_ref[...] = (y1_ref[...].astype(jnp.float32) * a_col).astype(jnp.bfloat16)


def _run_tc1(y1, hist):
    return pl.pallas_call(
        _tc1_body,
        out_shape=[
            jax.ShapeDtypeStruct((_N, _C), jnp.bfloat16),
            jax.ShapeDtypeStruct((_N, 1), jnp.float32),
        ],
    )(y1, hist)


def _tc2_body(aggp_ref, c_ref, z_ref, i_ref, wd_ref, bd_ref, o_ref):
    agg = aggp_ref[0].astype(jnp.float32) + aggp_ref[1].astype(jnp.float32)
    h = jax.nn.sigmoid(c_ref[...] * agg + z_ref[...].astype(jnp.float32))
    grow = lax.broadcasted_iota(jnp.int32, (_G, _N), 0)
    onehot_t = (grow == i_ref[...]).astype(jnp.bfloat16)
    sums = jnp.dot(onehot_t, h.astype(jnp.bfloat16),
                   preferred_element_type=jnp.float32)
    cnt = jnp.dot(onehot_t, jnp.ones((_N, 1), jnp.bfloat16),
                  preferred_element_type=jnp.float32)
    pooled = sums / jnp.maximum(cnt, 1.0)
    logits = jnp.dot(pooled, wd_ref[...],
                     preferred_element_type=jnp.float32) + bd_ref[...]
    m = jnp.max(logits, axis=1, keepdims=True)
    e = jnp.exp(logits - m)
    o_ref[...] = e / jnp.sum(e, axis=1, keepdims=True)


def _run_tc2(aggp, c_col, z, i_row, wd, bd2):
    return pl.pallas_call(
        _tc2_body,
        out_shape=jax.ShapeDtypeStruct((_G, _L), jnp.float32),
    )(aggp, c_col, z, i_row, wd, bd2)


def kernel(x, edge_index, i, W1, W2, b, Wd, bd):
    edge2 = edge_index.reshape(2, _NW, _KCH, _BCH)
    zeros = jnp.zeros((640, _C), jnp.bfloat16)
    b2 = b.reshape(1, _C)
    bd2 = bd.reshape(1, _L)
    i_row = i.reshape(1, _N)

    hist = _run_hist(edge2)
    y1, z = _run_tc0(x, W1, W2, b2)
    y1p, c_col = _run_tc1(y1, hist)
    aggp = _run_msg(y1p, edge2, zeros)
    return _run_tc2(aggp, c_col, z, i_row, Wd, bd2)


# speedup vs baseline: 38.6840x; 1.0866x over previous
import dataclasses

import jax
import jax.numpy as jnp
from jax import lax
from jax.experimental import pallas as pl
from jax.experimental.pallas import tpu as pltpu
from jax.experimental.pallas import tpu_sc as plsc

_N = 10000
_E = 160000
_D = 256
_C = 32
_G = 64
_L = 4

_NC = 2
_NS = 16
_NW = _NC * _NS
_EPW = _E // _NW
_KCH = 40
_BCH = _EPW // _KCH
_VL = 16
_NBUF = 8
_LOOK = 4


def _sc_params():
    cp = pltpu.CompilerParams(use_tc_tiling_on_sc=False)
    if "needs_layout_passes" in pltpu.CompilerParams.__dataclass_fields__:
        cp = dataclasses.replace(cp, needs_layout_passes=False)
    return cp


def _sc_mesh():
    return plsc.VectorSubcoreMesh(
        core_axis_name="core", subcore_axis_name="subcore",
        num_cores=_NC, num_subcores=_NS)


def _hist_body(edge_hbm, out_hbm, sidx, didx, hp):
    c = lax.axis_index("core")
    s = lax.axis_index("subcore")
    wid = c * _NS + s
    pltpu.sync_copy(edge_hbm.at[0].at[wid], sidx)
    pltpu.sync_copy(edge_hbm.at[1].at[wid], didx)

    z16 = jnp.zeros((_VL,), jnp.int32)

    @pl.loop(0, _N // _VL)
    def _(j):
        hp[pl.ds(j * _VL, _VL)] = z16

    def bump(idxv, weight, elig):
        cnt, last = plsc.scan_count(idxv, mask=elig)
        m = last if elig is None else (last & elig)
        plsc.addupdate_scatter(hp, [idxv], cnt * weight, mask=m)

    nfull = _BCH // _VL
    tail = nfull * _VL - (_BCH - _VL)
    elig = lax.iota(jnp.int32, _VL) >= tail

    @pl.loop(0, _KCH)
    def _(r):
        for w in range(nfull):
            bump(sidx[r, pl.ds(w * _VL, _VL)], 1, None)
            bump(didx[r, pl.ds(w * _VL, _VL)], 65536, None)
        bump(sidx[r, pl.ds(_BCH - _VL, _VL)], 1, elig)
        bump(didx[r, pl.ds(_BCH - _VL, _VL)], 65536, elig)

    pltpu.sync_copy(hp, out_hbm.at[wid])


def _run_hist(edge2):
    k = pl.kernel(
        _hist_body,
        out_type=jax.ShapeDtypeStruct((_NW, _N), jnp.int32),
        mesh=_sc_mesh(),
        compiler_params=_sc_params(),
        scratch_types=[
            pltpu.VMEM((_KCH, _BCH), jnp.int32),
            pltpu.VMEM((_KCH, _BCH), jnp.int32),
            pltpu.VMEM((_N,), jnp.int32),
        ],
    )
    return k(edge2)


def _msg_body(y_hbm, edge_hbm, zero_hbm, out_hbm,
              sidx, didx, rows, agg_sh, *sems):
    c = lax.axis_index("core")
    s = lax.axis_index("subcore")
    wid = c * _NS + s
    gsem = sems[:_NBUF]
    ssem = sems[_NBUF:]
    pltpu.sync_copy(edge_hbm.at[0].at[wid], sidx)
    pltpu.sync_copy(edge_hbm.at[1].at[wid], didx)
    slab = 640
    off = pl.multiple_of(jnp.minimum(s * slab, _N - slab), 8)
    pltpu.sync_copy(zero_hbm, agg_sh.at[pl.ds(off, slab)])
    plsc.subcore_barrier()

    def rslice(slot):
        return rows.at[pl.ds(slot * _BCH, _BCH)]

    def fire_gather(j, slot):
        pltpu.async_copy(y_hbm.at[sidx.at[j]], rslice(slot), gsem[slot])

    def wait_gather(j, slot):
        pltpu.make_async_copy(y_hbm.at[sidx.at[j]], rslice(slot),
                              gsem[slot]).wait()

    def fire_scatter(j, slot):
        pltpu.async_copy(rslice(slot), agg_sh.at[didx.at[j]], ssem[slot],
                         add=True)

    def wait_scatter(j, slot):
        pltpu.make_async_copy(rslice(slot), agg_sh.at[didx.at[j]],
                              ssem[slot]).wait()

    for p in range(_LOOK):
        fire_gather(p, p)

    @pl.loop(0, _KCH, step=_NBUF)
    def _(base):
        for b in range(_NBUF):
            j = base + b
            wait_gather(j, b)
            fire_scatter(j, b)
            nslot = (b + _LOOK) % _NBUF

            @pl.when(j + _LOOK < _KCH)
            def _():
                @pl.when(j - _LOOK >= 0)
                def _():
                    wait_scatter(j - _LOOK, nslot)
                fire_gather(j + _LOOK, nslot)

    for b in range(_NBUF):
        wait_scatter(_KCH - _NBUF + b, b)

    plsc.subcore_barrier()
    pltpu.sync_copy(agg_sh.at[pl.ds(off, slab)],
                    out_hbm.at[c].at[pl.ds(off, slab)])


def _run_msg(y1p, edge2, zeros):
    k = pl.kernel(
        _msg_body,
        out_type=jax.ShapeDtypeStruct((_NC, _N, _C), jnp.bfloat16),
        mesh=_sc_mesh(),
        compiler_params=_sc_params(),
        scratch_types=[
            pltpu.VMEM((_KCH, _BCH), jnp.int32),
            pltpu.VMEM((_KCH, _BCH), jnp.int32),
            pltpu.VMEM((_NBUF * _BCH, _C), jnp.bfloat16),
            pltpu.VMEM_SHARED((_N, _C), jnp.bfloat16),
        ] + [pltpu.SemaphoreType.DMA] * (2 * _NBUF),
    )
    return k(y1p, edge2, zeros)


def _tc0_body(x_ref, w1_ref, w2_ref, b_ref, y1_ref, z_ref):
    xb = x_ref[...].astype(jnp.bfloat16)
    y1_ref[...] = jnp.dot(xb, w1_ref[...].astype(jnp.bfloat16),
                          preferred_element_type=jnp.float32
                          ).astype(jnp.bfloat16)
    z_ref[...] = (jnp.dot(xb, w2_ref[...].astype(jnp.bfloat16),
                          preferred_element_type=jnp.float32)
                  + b_ref[...]).astype(jnp.bfloat16)


def _run_tc0(x, w1, w2, b2):
    return pl.pallas_call(
        _tc0_body,
        out_shape=[
            jax.ShapeDtypeStruct((_N, _C), jnp.bfloat16),
            jax.ShapeDtypeStruct((_N, _C), jnp.bfloat16),
        ],
    )(x, w1, w2, b2)


def _tc1_body(y1_ref, hist_ref, yp_ref):
    h = hist_ref[...]
    lo = (h & 0xFFFF).astype(jnp.float32)
    ones = jnp.ones((_NW, 1), jnp.float32)
    dego = lax.dot_general(lo, ones, (((0,), (0,)), ((), ())),
                           preferred_element_type=jnp.float32)
    a_col = lax.rsqrt(jnp.maximum(dego, 1.0))
    yp_ref[...] = (y1_ref[...].astype(jnp.float32) * a_col).astype(jnp.bfloat16)


def _run_tc1(y1, hist):
    return pl.pallas_call(
        _tc1_body,
        out_shape=jax.ShapeDtypeStruct((_N, _C), jnp.bfloat16),
    )(y1, hist)


def _tc2_body(aggp_ref, hist_ref, z_ref, i_ref, wd_ref, bd_ref, o_ref):
    hp = hist_ref[...]
    hi = lax.shift_right_logical(hp, 16).astype(jnp.float32)
    ones32 = jnp.ones((_NW, 1), jnp.float32)
    degi = lax.dot_general(hi, ones32, (((0,), (0,)), ((), ())),
                           preferred_element_type=jnp.float32)
    c_col = lax.rsqrt(jnp.maximum(degi, 1.0))
    agg = aggp_ref[0].astype(jnp.float32) + aggp_ref[1].astype(jnp.float32)
    h = jax.nn.sigmoid(c_col * agg + z_ref[...].astype(jnp.float32))
    grow = lax.broadcasted_iota(jnp.int32, (_G, _N), 0)
    onehot_t = (grow == i_ref[...]).astype(jnp.bfloat16)
    sums = jnp.dot(onehot_t, h.astype(jnp.bfloat16),
                   preferred_element_type=jnp.float32)
    cnt = jnp.dot(onehot_t, jnp.ones((_N, 1), jnp.bfloat16),
                  preferred_element_type=jnp.float32)
    pooled = sums / jnp.maximum(cnt, 1.0)
    logits = jnp.dot(pooled, wd_ref[...],
                     preferred_element_type=jnp.float32) + bd_ref[...]
    m = jnp.max(logits, axis=1, keepdims=True)
    e = jnp.exp(logits - m)
    o_ref[...] = e / jnp.sum(e, axis=1, keepdims=True)


def _run_tc2(aggp, hist, z, i_row, wd, bd2):
    return pl.pallas_call(
        _tc2_body,
        out_shape=jax.ShapeDtypeStruct((_G, _L), jnp.float32),
    )(aggp, hist, z, i_row, wd, bd2)


def kernel(x, edge_index, i, W1, W2, b, Wd, bd):
    edge2 = edge_index.reshape(2, _NW, _KCH, _BCH)
    zeros = jnp.zeros((640, _C), jnp.bfloat16)
    b2 = b.reshape(1, _C)
    bd2 = bd.reshape(1, _L)
    i_row = i.reshape(1, _N)

    hist = _run_hist(edge2)
    y1, z = _run_tc0(x, W1, W2, b2)
    y1p = _run_tc1(y1, hist)
    aggp = _run_msg(y1p, edge2, zeros)
    return _run_tc2(aggp, hist, z, i_row, Wd, bd2)
